# baseline (speedup 1.0000x reference)
import jax
import jax.numpy as jnp
from jax import lax
from jax.experimental import pallas as pl
from jax.experimental.pallas import tpu as pltpu
from jax.experimental.pallas import tpu_sc as plsc

BATCH = 16384
EMB_DIM = 32
NC = 2
NS = 16
NW = NC * NS
B_PER_W = BATCH // NW
CHUNK = 128
NCHUNK = B_PER_W // CHUNK
VROW = 4 * EMB_DIM
GROUPS = CHUNK // 16


def _body(uid_hbm, iid_hbm, uemb_hbm, iemb_hbm, out_hbm,
          uids_v, iids_v, vu_v, vi_v,
          ubuf0, ubuf1, ibuf0, ibuf1, out_v, sem0, sem1):
  wid = lax.axis_index("s") * NC + lax.axis_index("c")
  base = wid * B_PER_W

  for k in range(NCHUNK):
    pltpu.sync_copy(uid_hbm.at[pl.ds(base + k * CHUNK, CHUNK)], uids_v.at[k])
    pltpu.sync_copy(iid_hbm.at[pl.ds(base + k * CHUNK, CHUNK)], iids_v.at[k])

  for k in range(NCHUNK):
    for j in range(CHUNK // 16):
      s = pl.ds(j * 16, 16)
      vu_v[k, s] = lax.shift_right_logical(uids_v[k, s], 2)
      vi_v[k, s] = lax.shift_right_logical(iids_v[k, s], 2)

  ubufs = (ubuf0, ubuf1)
  ibufs = (ibuf0, ibuf1)
  sems = (sem0, sem1)

  def fire(k):
    s = sems[k % 2]
    return (pltpu.async_copy(uemb_hbm.at[vu_v.at[k]], ubufs[k % 2], s),
            pltpu.async_copy(iemb_hbm.at[vi_v.at[k]], ibufs[k % 2], s))

  lane = jnp.arange(16, dtype=jnp.int32)
  inflight = fire(0)

  for k in range(NCHUNK):
    for c in inflight:
      c.wait()
    if k + 1 < NCHUNK:
      inflight = fire(k + 1)
    ub = ubufs[k % 2]
    ib = ibufs[k % 2]

    def group(g, _):
      rid = g * 16 + lane
      su = (uids_v[k, pl.ds(g * 16, 16)] & 3) * EMB_DIM
      si = (iids_v[k, pl.ds(g * 16, 16)] & 3) * EMB_DIM
      acc = jnp.zeros((16,), jnp.float32)
      for d in range(EMB_DIM):
        uv = plsc.load_gather(ub, [rid, su + d])
        iv = plsc.load_gather(ib, [rid, si + d])
        acc = acc + uv * iv
      sig = 1.0 / (1.0 + jnp.exp(-acc))
      plsc.store_scatter(out_v, [k * CHUNK + rid], sig)
      return _

    lax.fori_loop(0, GROUPS, group, None)

  pltpu.sync_copy(out_v, out_hbm.at[pl.ds(base, B_PER_W)])


@jax.jit
def _run(user_ids, item_ids, user_emb, item_emb):
  mesh = plsc.VectorSubcoreMesh(core_axis_name="c", subcore_axis_name="s")
  k = pl.kernel(
      _body,
      out_type=jax.ShapeDtypeStruct((BATCH,), jnp.float32),
      mesh=mesh,
      compiler_params=pltpu.CompilerParams(needs_layout_passes=False),
      scratch_types=[
          pltpu.VMEM((NCHUNK, CHUNK), jnp.int32),
          pltpu.VMEM((NCHUNK, CHUNK), jnp.int32),
          pltpu.VMEM((NCHUNK, CHUNK), jnp.int32),
          pltpu.VMEM((NCHUNK, CHUNK), jnp.int32),
          pltpu.VMEM((CHUNK, VROW), jnp.float32),
          pltpu.VMEM((CHUNK, VROW), jnp.float32),
          pltpu.VMEM((CHUNK, VROW), jnp.float32),
          pltpu.VMEM((CHUNK, VROW), jnp.float32),
          pltpu.VMEM((B_PER_W,), jnp.float32),
          pltpu.SemaphoreType.DMA,
          pltpu.SemaphoreType.DMA,
      ],
  )
  uview = user_emb.reshape(-1, VROW)
  iview = item_emb.reshape(-1, VROW)
  return k(user_ids, item_ids, uview, iview)


def kernel(user_ids, item_ids, user_emb, item_emb):
  return _run(user_ids.astype(jnp.int32), item_ids.astype(jnp.int32),
              user_emb, item_emb)


# speedup vs baseline: 1.0067x; 1.0067x over previous
import jax
import jax.numpy as jnp
from jax import lax
from jax.experimental import pallas as pl
from jax.experimental.pallas import tpu as pltpu
from jax.experimental.pallas import tpu_sc as plsc

BATCH = 16384
EMB_DIM = 32
NC = 2
NS = 16
NW = NC * NS
B_PER_W = BATCH // NW
CHUNK = 128
NCHUNK = B_PER_W // CHUNK


def _gather_body(ids_hbm, emb_hbm, out_hbm, ids_v, rows_v, sem):
  wid = lax.axis_index("s") * NC + lax.axis_index("c")
  base = wid * B_PER_W
  for k in range(NCHUNK):
    pltpu.sync_copy(ids_hbm.at[pl.ds(base + k * CHUNK, CHUNK)], ids_v.at[k])
  copies = []
  for k in range(NCHUNK):
    copies.append(pltpu.async_copy(
        emb_hbm.at[ids_v.at[k]], rows_v.at[pl.ds(k * CHUNK, CHUNK)], sem))
  for c in copies:
    c.wait()
  pltpu.sync_copy(rows_v, out_hbm.at[pl.ds(base, B_PER_W), :])


def _dot_body(iid_hbm, iemb_hbm, urows_hbm, out_hbm,
              iids_v, urows_v, irows_v, out_v, sem):
  wid = lax.axis_index("s") * NC + lax.axis_index("c")
  base = wid * B_PER_W
  for k in range(NCHUNK):
    pltpu.sync_copy(iid_hbm.at[pl.ds(base + k * CHUNK, CHUNK)], iids_v.at[k])
  copies = [pltpu.async_copy(urows_hbm.at[pl.ds(base, B_PER_W), :],
                             urows_v, sem)]
  for k in range(NCHUNK):
    copies.append(pltpu.async_copy(
        iemb_hbm.at[iids_v.at[k]], irows_v.at[pl.ds(k * CHUNK, CHUNK)], sem))
  for c in copies:
    c.wait()

  lane = jnp.arange(16, dtype=jnp.int32)

  def group(g, _):
    rid = g * 16 + lane
    acc = jnp.zeros((16,), jnp.float32)
    for d in range(EMB_DIM):
      dd = jnp.full((16,), d, jnp.int32)
      uv = plsc.load_gather(urows_v, [rid, dd])
      iv = plsc.load_gather(irows_v, [rid, dd])
      acc = acc + uv * iv
    sig = 1.0 / (1.0 + jnp.exp(-acc))
    plsc.store_scatter(out_v, [rid], sig)
    return _

  lax.fori_loop(0, B_PER_W // 16, group, None)
  pltpu.sync_copy(out_v, out_hbm.at[pl.ds(base, B_PER_W)])


@jax.jit
def _run(user_ids, item_ids, user_emb, item_emb):
  mesh = plsc.VectorSubcoreMesh(core_axis_name="c", subcore_axis_name="s")
  params = pltpu.CompilerParams(
      needs_layout_passes=False, use_tc_tiling_on_sc=False)
  gather_u = pl.kernel(
      _gather_body,
      out_type=jax.ShapeDtypeStruct((BATCH, EMB_DIM), jnp.float32),
      mesh=mesh,
      compiler_params=params,
      scratch_types=[
          pltpu.VMEM((NCHUNK, CHUNK), jnp.int32),
          pltpu.VMEM((B_PER_W, EMB_DIM), jnp.float32),
          pltpu.SemaphoreType.DMA,
      ],
  )
  dot = pl.kernel(
      _dot_body,
      out_type=jax.ShapeDtypeStruct((BATCH,), jnp.float32),
      mesh=mesh,
      compiler_params=params,
      scratch_types=[
          pltpu.VMEM((NCHUNK, CHUNK), jnp.int32),
          pltpu.VMEM((B_PER_W, EMB_DIM), jnp.float32),
          pltpu.VMEM((B_PER_W, EMB_DIM), jnp.float32),
          pltpu.VMEM((B_PER_W,), jnp.float32),
          pltpu.SemaphoreType.DMA,
      ],
  )
  urows = gather_u(user_ids, user_emb)
  return dot(item_ids, item_emb, urows)


def kernel(user_ids, item_ids, user_emb, item_emb):
  return _run(user_ids.astype(jnp.int32), item_ids.astype(jnp.int32),
              user_emb, item_emb)
